# baseline (speedup 1.0000x reference)
import jax
import jax.numpy as jnp
from jax import lax
from jax.experimental import pallas as pl
from jax.experimental.pallas import tpu as pltpu
from jax.experimental.pallas import tpu_sc as plsc

N_NEG = 20
HIST = 50
DIM = 32
BATA = 0.01
LAMDA = 0.01
ALPHA = 0.5
GAMA = 0.1

L = 16
NC = 2
NS = 16
NW = NC * NS


def _rsqrt(x):
    i = plsc.bitcast(x, jnp.int32)
    i = jnp.int32(0x5F3759DF) - (i >> 1)
    y = plsc.bitcast(i, jnp.float32)
    for _ in range(3):
        y = y * (1.5 - 0.5 * x * y * y)
    return y


def _full(v):
    return jnp.full((L,), v, jnp.int32)


def _sc_body(bu, bi, qi, pu, users, pos_items, neg_items, uin, interacted,
             out, users_v, pos_v, neg2d, uin_v, negidx, histidx, histval,
             bipos_v, bineg_v, buv_v, pu_buf, ue, qip, negbuf, stage, sem):
    wid = lax.axis_index("s") * NC + lax.axis_index("c")
    U = 128
    base = wid * U
    zero = jnp.zeros((L,), jnp.float32)
    iota = lax.iota(jnp.int32, L)

    pltpu.sync_copy(users.at[pl.ds(base, U)], users_v)
    pltpu.sync_copy(pos_items.at[pl.ds(base, U)], pos_v)
    pltpu.sync_copy(neg_items.at[pl.ds(base, U)], neg2d)
    pltpu.sync_copy(uin.at[pl.ds(base, U)], uin_v)

    pltpu.async_copy(qi.at[pos_v], qip, sem).wait()
    pltpu.async_copy(bi.at[pos_v], bipos_v, sem).wait()
    pltpu.async_copy(bu.at[users_v], buv_v, sem).wait()

    for g in range(U // L):
        uvec = iota + g * L
        def _nbody(j, _):
            v = plsc.load_gather(neg2d, [uvec, _full(j)])
            negidx[pl.ds(g * (L * N_NEG) + j * L, L)] = v
            return 0
        lax.fori_loop(0, N_NEG, _nbody, 0)
    pltpu.async_copy(bi.at[negidx], bineg_v, sem).wait()

    acc = zero
    acc2 = zero
    acc3 = zero
    acc4 = zero

    for c in range(U // L):
        uv50 = users_v[pl.ds(c * L, L)] * HIST
        def _fbody(h, _):
            histidx[pl.ds(h * L, L)] = uv50 + h
            return 0
        lax.fori_loop(0, HIST, _fbody, 0)
        pltpu.async_copy(interacted.at[histidx], histval, sem).wait()
        pltpu.async_copy(pu.at[histval], pu_buf, sem).wait()
        for u in range(L):
            def _hbody(h, carry):
                a0, a1 = carry
                r = h * L + u
                return (a0 + pu_buf[r, pl.ds(0, L)],
                        a1 + pu_buf[r, pl.ds(L, L)])
            a0, a1 = lax.fori_loop(0, HIST, _hbody, (zero, zero))
            col = c * L + u
            ue[col, pl.ds(0, L)] = a0
            ue[col, pl.ds(L, L)] = a1
            acc2 = acc2 + a0 * a0 + a1 * a1

    for g in range(U // L):
        pltpu.async_copy(
            qi.at[negidx.at[pl.ds(g * (L * N_NEG), L * N_NEG)]], negbuf, sem
        ).wait()
        uvec = iota + g * L
        t = _rsqrt(uin_v[pl.ds(g * L, L)].astype(jnp.float32))
        bu_g = buv_v[pl.ds(g * L, L)]
        bp_g = bipos_v[pl.ds(g * L, L)]

        def _pbody(d, carry):
            dot, sq = carry
            ue_d = plsc.load_gather(ue, [uvec, _full(d)])
            qp_d = plsc.load_gather(qip, [uvec, _full(d)])
            return dot + ue_d * qp_d, sq + qp_d * qp_d
        dot, sq = lax.fori_loop(0, DIM, _pbody, (zero, zero))
        acc2 = acc2 + sq
        z = t * dot + bp_g + bu_g
        sp = 1.0 / (1.0 + jnp.exp(z))
        acc = acc + sp * sp
        acc3 = acc3 + bp_g * bp_g
        acc4 = acc4 + bu_g * bu_g

        def _jbody(j, acc_in):
            jrow = iota + j * L
            def _dbody(d, dotn):
                ue_d = plsc.load_gather(ue, [uvec, _full(d)])
                nv = plsc.load_gather(negbuf, [jrow, _full(d)])
                return dotn + ue_d * nv
            dotn = lax.fori_loop(0, DIM, _dbody, zero)
            bj = bineg_v[pl.ds(g * (L * N_NEG) + j * L, L)]
            zj = t * dotn + bj + bu_g
            sn = 1.0 / (1.0 + jnp.exp(-zj))
            return acc_in + sn * sn
        acc = lax.fori_loop(0, N_NEG, _jbody, acc)

    total = acc + BATA * acc2 + LAMDA * acc3 + (LAMDA * GAMA) * acc4
    stage[...] = total
    pltpu.sync_copy(stage, out.at[wid])


@jax.jit
def _fism_loss(bu, bi, qi, pu, users, pos_items, neg_items, uin, interacted):
    U = 128
    kern = pl.kernel(
        _sc_body,
        out_type=jax.ShapeDtypeStruct((NW, L), jnp.float32),
        mesh=plsc.VectorSubcoreMesh(core_axis_name="c", subcore_axis_name="s"),
        compiler_params=pltpu.CompilerParams(
            needs_layout_passes=False, use_tc_tiling_on_sc=False),
        scratch_types=[
            pltpu.VMEM((U,), jnp.int32),
            pltpu.VMEM((U,), jnp.int32),
            pltpu.VMEM((U, N_NEG), jnp.int32),
            pltpu.VMEM((U,), jnp.int32),
            pltpu.VMEM((U * N_NEG,), jnp.int32),
            pltpu.VMEM((L * HIST,), jnp.int32),
            pltpu.VMEM((L * HIST,), jnp.int32),
            pltpu.VMEM((U,), jnp.float32),
            pltpu.VMEM((U * N_NEG,), jnp.float32),
            pltpu.VMEM((U,), jnp.float32),
            pltpu.VMEM((L * HIST, DIM), jnp.float32),
            pltpu.VMEM((U, DIM), jnp.float32),
            pltpu.VMEM((U, DIM), jnp.float32),
            pltpu.VMEM((L * N_NEG, DIM), jnp.float32),
            pltpu.VMEM((L,), jnp.float32),
            pltpu.SemaphoreType.DMA,
        ],
    )
    partials = kern(bu, bi, qi, pu, users, pos_items, neg_items, uin,
                    interacted)
    return jnp.sum(partials)


def kernel(bu, bi, qi, pu, users, pos_items, neg_items, user_item_num,
           interacted_items):
    return _fism_loss(bu.reshape(-1), bi.reshape(-1), qi, pu, users,
                      pos_items, neg_items, user_item_num,
                      interacted_items.reshape(-1))


# speedup vs baseline: 1.0941x; 1.0941x over previous
import jax
import jax.numpy as jnp
from jax import lax
from jax.experimental import pallas as pl
from jax.experimental.pallas import tpu as pltpu
from jax.experimental.pallas import tpu_sc as plsc

N_NEG = 20
HIST = 50
DIM = 32
BATA = 0.01
LAMDA = 0.01
ALPHA = 0.5
GAMA = 0.1

L = 16
NC = 2
NS = 16
NW = NC * NS
U = 128
NG = U // L


def _rsqrt(x):
    i = plsc.bitcast(x, jnp.int32)
    i = jnp.int32(0x5F3759DF) - (i >> 1)
    y = plsc.bitcast(i, jnp.float32)
    for _ in range(3):
        y = y * (1.5 - 0.5 * x * y * y)
    return y


def _full(v):
    return jnp.full((L,), v, jnp.int32)


def _sc_body(bu, bi, qi, pu, users, pos_items, neg_flat, uin, inter16,
             out, users_v, pos_v, uin_v, negidx, winidx, histwin, histval,
             bipos_v, bineg_v, buv_v, pu_buf0, pu_buf1, ue, qip,
             negbuf0, negbuf1, stage,
             semq, semb1, semb2, semb3, semw,
             semA0, semA1, semN0, semN1):
    wid = lax.axis_index("s") * NC + lax.axis_index("c")
    base = wid * U
    zero = jnp.zeros((L,), jnp.float32)
    iota = lax.iota(jnp.int32, L)

    pltpu.sync_copy(users.at[pl.ds(base, U)], users_v)
    pltpu.sync_copy(pos_items.at[pl.ds(base, U)], pos_v)
    pltpu.sync_copy(neg_flat.at[pl.ds(base * N_NEG, U * N_NEG)], negidx)
    pltpu.sync_copy(uin.at[pl.ds(base, U)], uin_v)

    cp_qip = pltpu.async_copy(qi.at[pos_v], qip, semq)
    cp_bip = pltpu.async_copy(bi.at[pos_v], bipos_v, semb1)
    cp_buv = pltpu.async_copy(bu.at[users_v], buv_v, semb2)
    cp_bin = pltpu.async_copy(bi.at[negidx], bineg_v, semb3)

    for c in range(NG):
        b = (users_v[pl.ds(c * L, L)] * HIST) >> 4
        for k in range(4):
            plsc.store_scatter(winidx, [iota * 4 + (c * 64 + k)], b + k)
    cp_win = pltpu.async_copy(inter16.at[winidx], histwin, semw)
    cp_win.wait()

    for c in range(NG):
        off = (users_v[pl.ds(c * L, L)] * HIST) & 15
        fbase = off + (c * 1024) + iota * 64

        def _ebody(h, _):
            flat = fbase + h
            v = plsc.load_gather(histwin, [flat >> 4, flat & 15])
            histval[pl.ds(c * (L * HIST) + h * L, L)] = v
            return 0
        lax.fori_loop(0, HIST, _ebody, 0)

    acc = zero
    acc2 = zero
    acc3 = zero
    acc4 = zero

    pu_bufs = (pu_buf0, pu_buf1)
    semsA = (semA0, semA1)
    cps = [None, None]
    cps[0] = pltpu.async_copy(
        pu.at[histval.at[pl.ds(0, L * HIST)]], pu_buf0, semA0)
    for c in range(NG):
        par = c % 2
        cps[par].wait()
        if c + 1 < NG:
            nxt = (c + 1) % 2
            cps[nxt] = pltpu.async_copy(
                pu.at[histval.at[pl.ds((c + 1) * (L * HIST), L * HIST)]],
                pu_bufs[nxt], semsA[nxt])
        buf = pu_bufs[par]

        def _ubody(u, acc2_in):
            a0 = buf[u, pl.ds(0, L)]
            a1 = buf[u, pl.ds(L, L)]
            for h in range(1, HIST):
                r = h * L + u
                a0 = a0 + buf[r, pl.ds(0, L)]
                a1 = a1 + buf[r, pl.ds(L, L)]
            col = c * L + u
            ue[col, pl.ds(0, L)] = a0
            ue[col, pl.ds(L, L)] = a1
            return acc2_in + a0 * a0 + a1 * a1
        acc2 = lax.fori_loop(0, L, _ubody, acc2)

    cp_qip.wait()
    cp_bip.wait()
    cp_buv.wait()
    cp_bin.wait()
    neg_bufs = (negbuf0, negbuf1)
    semsN = (semN0, semN1)
    cpn = [None, None]
    cpn[0] = pltpu.async_copy(
        qi.at[negidx.at[pl.ds(0, L * N_NEG)]], negbuf0, semN0)
    for g in range(NG):
        par = g % 2
        cpn[par].wait()
        if g + 1 < NG:
            nxt = (g + 1) % 2
            cpn[nxt] = pltpu.async_copy(
                qi.at[negidx.at[pl.ds((g + 1) * (L * N_NEG), L * N_NEG)]],
                neg_bufs[nxt], semsN[nxt])
        nbuf = neg_bufs[par]

        uvec = iota + g * L
        t = _rsqrt(uin_v[pl.ds(g * L, L)].astype(jnp.float32))
        bu_g = buv_v[pl.ds(g * L, L)]
        bp_g = bipos_v[pl.ds(g * L, L)]

        ue_cols = [plsc.load_gather(ue, [uvec, _full(d)]) for d in range(DIM)]

        dot = zero
        sq = zero
        for d in range(DIM):
            qp_d = plsc.load_gather(qip, [uvec, _full(d)])
            dot = dot + ue_cols[d] * qp_d
            sq = sq + qp_d * qp_d
        acc2 = acc2 + sq
        z = t * dot + bp_g + bu_g
        sp = 1.0 / (1.0 + jnp.exp(z))
        acc = acc + sp * sp
        acc3 = acc3 + bp_g * bp_g
        acc4 = acc4 + bu_g * bu_g

        jvec = iota * N_NEG + g * (L * N_NEG)

        def _jbody(j, acc_in):
            rows = iota * N_NEG + j
            dotn = zero
            for d in range(DIM):
                nv = plsc.load_gather(nbuf, [rows, _full(d)])
                dotn = dotn + ue_cols[d] * nv
            bj = plsc.load_gather(bineg_v, [jvec + j])
            zj = t * dotn + bj + bu_g
            sn = 1.0 / (1.0 + jnp.exp(-zj))
            return acc_in + sn * sn
        acc = lax.fori_loop(0, N_NEG, _jbody, acc)

    total = acc + BATA * acc2 + LAMDA * acc3 + (LAMDA * GAMA) * acc4
    stage[...] = total
    pltpu.sync_copy(stage, out.at[wid])


@jax.jit
def _fism_loss(bu, bi, qi, pu, users, pos_items, neg_flat, uin, inter16):
    kern = pl.kernel(
        _sc_body,
        out_type=jax.ShapeDtypeStruct((NW, L), jnp.float32),
        mesh=plsc.VectorSubcoreMesh(core_axis_name="c", subcore_axis_name="s"),
        compiler_params=pltpu.CompilerParams(
            needs_layout_passes=False, use_tc_tiling_on_sc=False),
        scratch_types=[
            pltpu.VMEM((U,), jnp.int32),
            pltpu.VMEM((U,), jnp.int32),
            pltpu.VMEM((U,), jnp.int32),
            pltpu.VMEM((U * N_NEG,), jnp.int32),
            pltpu.VMEM((U * 4,), jnp.int32),
            pltpu.VMEM((U * 4, L), jnp.int32),
            pltpu.VMEM((U * HIST,), jnp.int32),
            pltpu.VMEM((U,), jnp.float32),
            pltpu.VMEM((U * N_NEG,), jnp.float32),
            pltpu.VMEM((U,), jnp.float32),
            pltpu.VMEM((L * HIST, DIM), jnp.float32),
            pltpu.VMEM((L * HIST, DIM), jnp.float32),
            pltpu.VMEM((U, DIM), jnp.float32),
            pltpu.VMEM((U, DIM), jnp.float32),
            pltpu.VMEM((L * N_NEG, DIM), jnp.float32),
            pltpu.VMEM((L * N_NEG, DIM), jnp.float32),
            pltpu.VMEM((L,), jnp.float32),
            pltpu.SemaphoreType.DMA,
            pltpu.SemaphoreType.DMA,
            pltpu.SemaphoreType.DMA,
            pltpu.SemaphoreType.DMA,
            pltpu.SemaphoreType.DMA,
            pltpu.SemaphoreType.DMA,
            pltpu.SemaphoreType.DMA,
            pltpu.SemaphoreType.DMA,
            pltpu.SemaphoreType.DMA,
        ],
    )
    partials = kern(bu, bi, qi, pu, users, pos_items, neg_flat, uin, inter16)
    return jnp.sum(partials)


def kernel(bu, bi, qi, pu, users, pos_items, neg_items, user_item_num,
           interacted_items):
    n_user, hist = interacted_items.shape
    return _fism_loss(bu.reshape(-1), bi.reshape(-1), qi, pu, users,
                      pos_items, neg_items.reshape(-1), user_item_num,
                      interacted_items.reshape(n_user * hist // L, L))
